# baseline (speedup 1.0000x reference)
import functools

import jax
import jax.numpy as jnp
from jax import lax
from jax.experimental import pallas as pl
from jax.experimental.pallas import tpu as pltpu
from jax.experimental.pallas import tpu_sc as plsc

_LN_EPS = 1e-5
_NC = 2
_NS = 16
_NW = _NC * _NS
_IDXW = 128


def _gather_body(nchunks, chunk, ids_hbm, table_hbm, out_hbm, idx_v, rows_v, sem):
    wid = lax.axis_index("s") * _NC + lax.axis_index("c")
    nsub = chunk // _IDXW
    rows_per_w = nchunks * chunk
    base_row = wid * rows_per_w

    def step(i, carry):
        off = pl.multiple_of(base_row + i * chunk, chunk)
        pltpu.sync_copy(ids_hbm.at[pl.ds(pl.multiple_of(off // _IDXW, nsub), nsub)], idx_v)
        copies = []
        for j in range(nsub):
            cp = pltpu.make_async_copy(
                table_hbm.at[idx_v.at[j]],
                rows_v.at[pl.ds(j * _IDXW, _IDXW)],
                sem,
            )
            cp.start()
            copies.append(cp)
        for cp in copies:
            cp.wait()
        pltpu.sync_copy(rows_v, out_hbm.at[pl.ds(off, chunk)])
        return carry

    lax.fori_loop(0, nchunks, step, 0)


def _sc_gather(ids2d, table, nchunks, chunk):
    n = ids2d.shape[0] * _IDXW
    d = table.shape[1]
    nsub = chunk // _IDXW
    mesh = plsc.VectorSubcoreMesh(core_axis_name="c", subcore_axis_name="s")
    f = pl.kernel(
        functools.partial(_gather_body, nchunks, chunk),
        out_type=jax.ShapeDtypeStruct((n, d), jnp.float32),
        mesh=mesh,
        scratch_types=[
            pltpu.VMEM((nsub, _IDXW), jnp.int32),
            pltpu.VMEM((chunk, d), jnp.float32),
            pltpu.SemaphoreType.DMA,
        ],
        compiler_params=pltpu.CompilerParams(use_tc_tiling_on_sc=False),
    )
    return f(ids2d, table)


def _lin_ln_body(w_ref, b_ref, g_ref, be_ref, emb_ref, out_ref):
    x = emb_ref[...]
    w = w_ref[...]
    y = lax.dot_general(x, w, (((1,), (1,)), ((), ())),
                        preferred_element_type=jnp.float32)
    y = y + b_ref[...]
    m = jnp.mean(y, axis=1, keepdims=True)
    c = y - m
    v = jnp.mean(c * c, axis=1, keepdims=True)
    out_ref[...] = (c * lax.rsqrt(v + _LN_EPS)) * g_ref[...] + be_ref[...]


def _lin_ln(emb, W, b, gamma, beta, blk):
    n, d = emb.shape
    return pl.pallas_call(
        _lin_ln_body,
        grid=(n // blk,),
        in_specs=[
            pl.BlockSpec((d, d), lambda i: (0, 0)),
            pl.BlockSpec((1, d), lambda i: (0, 0)),
            pl.BlockSpec((1, d), lambda i: (0, 0)),
            pl.BlockSpec((1, d), lambda i: (0, 0)),
            pl.BlockSpec((blk, d), lambda i: (i, 0)),
        ],
        out_specs=pl.BlockSpec((blk, d), lambda i: (i, 0)),
        out_shape=jax.ShapeDtypeStruct((n, d), jnp.float32),
    )(W, b.reshape(1, d), gamma.reshape(1, d), beta.reshape(1, d), emb)


def kernel(concept_ids, table, W, b, gamma, beta):
    bsz, fields = concept_ids.shape
    d = table.shape[1]
    n = bsz * fields

    chunk = 1024
    per = _NW * chunk
    n_pad = ((n + per - 1) // per) * per
    ids = concept_ids.reshape(n).astype(jnp.int32)
    if n_pad != n:
        ids = jnp.concatenate([ids, jnp.zeros((n_pad - n,), jnp.int32)])
    ids2d = ids.reshape(n_pad // _IDXW, _IDXW)

    emb = _sc_gather(ids2d, table, n_pad // per, chunk)
    if n_pad != n:
        emb = emb[:n]

    out = _lin_ln(emb, W, b, gamma, beta, blk=2048)
    return out.reshape(bsz, fields, d)


# speedup vs baseline: 1.3617x; 1.3617x over previous
import functools

import jax
import jax.numpy as jnp
from jax import lax
from jax.experimental import pallas as pl
from jax.experimental.pallas import tpu as pltpu
from jax.experimental.pallas import tpu_sc as plsc

_LN_EPS = 1e-5
_NC = 2
_NS = 16
_NW = _NC * _NS
_IDXW = 128


def _gather_body(nchunks, chunk, ids_hbm, table_hbm, out_hbm, idx_v, rows_v, sem):
    wid = lax.axis_index("s") * _NC + lax.axis_index("c")
    nsub = chunk // _IDXW
    rows_per_w = nchunks * chunk
    base_row = wid * rows_per_w

    def step(i, carry):
        off = pl.multiple_of(base_row + i * chunk, chunk)
        pltpu.sync_copy(ids_hbm.at[pl.ds(pl.multiple_of(off // _IDXW, nsub), nsub)], idx_v)
        copies = []
        for j in range(nsub):
            cp = pltpu.make_async_copy(
                table_hbm.at[idx_v.at[j]],
                rows_v.at[pl.ds(j * _IDXW, _IDXW)],
                sem,
            )
            cp.start()
            copies.append(cp)
        for cp in copies:
            cp.wait()
        pltpu.sync_copy(rows_v, out_hbm.at[pl.ds(off, chunk)])
        return carry

    lax.fori_loop(0, nchunks, step, 0)


def _sc_gather(ids2d, table, nchunks, chunk):
    n = ids2d.shape[0] * _IDXW
    d = table.shape[1]
    nsub = chunk // _IDXW
    mesh = plsc.VectorSubcoreMesh(core_axis_name="c", subcore_axis_name="s")
    f = pl.kernel(
        functools.partial(_gather_body, nchunks, chunk),
        out_type=jax.ShapeDtypeStruct((n, d), jnp.float32),
        mesh=mesh,
        scratch_types=[
            pltpu.VMEM((nsub, _IDXW), jnp.int32),
            pltpu.VMEM((chunk, d), jnp.float32),
            pltpu.SemaphoreType.DMA,
        ],
        compiler_params=pltpu.CompilerParams(use_tc_tiling_on_sc=False),
    )
    return f(ids2d, table)


def _lin_ln_t_body(w_ref, b_ref, g_ref, be_ref, emb_ref, out_ref):
    x = emb_ref[...]
    w = w_ref[...]
    y = lax.dot_general(w, x, (((1,), (1,)), ((), ())),
                        preferred_element_type=jnp.float32)
    y = y + b_ref[...]
    m = jnp.mean(y, axis=0, keepdims=True)
    c = y - m
    v = jnp.mean(c * c, axis=0, keepdims=True)
    r = (c * lax.rsqrt(v + _LN_EPS)) * g_ref[...] + be_ref[...]
    out_ref[...] = r[None]


def _lin_ln_t(emb, W, b, gamma, beta, fields, bsz, blk):
    d = emb.shape[1]
    nb = bsz // blk
    return pl.pallas_call(
        _lin_ln_t_body,
        grid=(fields, nb),
        in_specs=[
            pl.BlockSpec((d, d), lambda f, i: (0, 0)),
            pl.BlockSpec((d, 1), lambda f, i: (0, 0)),
            pl.BlockSpec((d, 1), lambda f, i: (0, 0)),
            pl.BlockSpec((d, 1), lambda f, i: (0, 0)),
            pl.BlockSpec((blk, d), lambda f, i: (f * nb + i, 0)),
        ],
        out_specs=pl.BlockSpec((1, d, blk), lambda f, i: (f, 0, i)),
        out_shape=jax.ShapeDtypeStruct((fields, d, bsz), jnp.float32),
    )(W, b.reshape(d, 1), gamma.reshape(d, 1), beta.reshape(d, 1), emb)


def kernel(concept_ids, table, W, b, gamma, beta):
    bsz, fields = concept_ids.shape
    d = table.shape[1]
    n = bsz * fields

    chunk = 1024
    per = _NW * chunk
    n_pad = ((n + per - 1) // per) * per
    ids = concept_ids.T.reshape(n).astype(jnp.int32)
    if n_pad != n:
        ids = jnp.concatenate([ids, jnp.zeros((n_pad - n,), jnp.int32)])
    ids2d = ids.reshape(n_pad // _IDXW, _IDXW)

    emb = _sc_gather(ids2d, table, n_pad // per, chunk)
    if n_pad != n:
        emb = emb[:n]

    out_t = _lin_ln_t(emb, W, b, gamma, beta, fields, bsz, blk=2048)
    return out_t.transpose(2, 0, 1)
